# baseline (speedup 1.0000x reference)
import functools

import jax
import jax.numpy as jnp
from jax import lax
from jax.experimental import pallas as pl
from jax.experimental.pallas import tpu as pltpu
from jax.experimental.pallas import tpu_sc as plsc

B = 16384
F = 100
CARD = 10000
D_FEAT = 28
D_SH = 4
D = 32
V = F * CARD + 1
N = B * F
NC, NS, L = 2, 16, 16
NW = NC * NS

ROWS_W = N // NW
CHUNK = 400
NCHUNK = ROWS_W // CHUNK
SUB = 100
NSUB = CHUNK // SUB

BCHUNK = 1600
NBCHUNK = (V - 1) // BCHUNK
BITER = (NBCHUNK + NW - 1) // NW
SB = 400


def _build_table_kernel():
    mesh = plsc.VectorSubcoreMesh(core_axis_name="c", subcore_axis_name="s")

    @functools.partial(
        pl.kernel,
        mesh=mesh,
        out_type=jax.ShapeDtypeStruct((V * D,), jnp.float32),
        compiler_params=pltpu.CompilerParams(use_tc_tiling_on_sc=False),
        scratch_types=[
            pltpu.VMEM((BCHUNK * D_FEAT,), jnp.float32),
            pltpu.VMEM((BCHUNK * D,), jnp.float32),
            pltpu.VMEM((F * D_SH + L,), jnp.float32),
            pltpu.SemaphoreType.DMA,
        ],
    )
    def k(feat_hbm, sh_hbm, ptab_hbm, fin_v, buf_v, sh_v, sem):
        wid = lax.axis_index("s") * NC + lax.axis_index("c")
        pltpu.sync_copy(sh_hbm, sh_v)

        def do_chunk(c):
            base = c * BCHUNK
            pltpu.sync_copy(
                feat_hbm.at[pl.ds(base * D_FEAT, BCHUNK * D_FEAT)], fin_v
            )
            for sb in range(BCHUNK // SB):
                t0 = base + sb * SB
                f_rest = t0 // CARD
                h = sh_v[pl.ds(D_SH * f_rest, L)]

                def row4(i, carry):
                    r = sb * SB + i * 4
                    for u in range(4):
                        o = (r + u) * D
                        g = (r + u) * D_FEAT
                        buf_v[pl.ds(o, L)] = h
                        buf_v[pl.ds(o + D_SH, L)] = fin_v[pl.ds(g, L)]
                        buf_v[pl.ds(o + D_SH + D_FEAT - L, L)] = fin_v[
                            pl.ds(g + D_FEAT - L, L)
                        ]
                    return carry

                lax.fori_loop(0, SB // 4, row4, 0)
                f_first = jnp.maximum(t0 - 1, 0) // CARD
                h0 = sh_v[pl.ds(D_SH * f_first, L)]
                buf_v[pl.ds(sb * SB * D, L)] = h0
                buf_v[pl.ds(sb * SB * D + D_SH, L)] = fin_v[
                    pl.ds(sb * SB * D_FEAT, L)
                ]
            pltpu.sync_copy(buf_v, ptab_hbm.at[pl.ds(base * D, BCHUNK * D)])

        def body(i, carry):
            c = wid + NW * i

            @pl.when(c < NBCHUNK)
            def _():
                do_chunk(c)

            return carry

        lax.fori_loop(0, BITER, body, 0)

        @pl.when(wid == NW - 1)
        def _():
            t = V - 1
            pltpu.sync_copy(
                feat_hbm.at[pl.ds(t * D_FEAT, D_FEAT)], fin_v.at[pl.ds(0, D_FEAT)]
            )
            buf_v[pl.ds(0, L)] = sh_v[pl.ds(D_SH * (F - 1), L)]
            buf_v[pl.ds(D_SH, L)] = fin_v[pl.ds(0, L)]
            buf_v[pl.ds(D_SH + D_FEAT - L, L)] = fin_v[pl.ds(D_FEAT - L, L)]
            pltpu.sync_copy(buf_v.at[pl.ds(0, D)], ptab_hbm.at[pl.ds(t * D, D)])

    return k


def _gather_phase_kernel():
    mesh = plsc.VectorSubcoreMesh(core_axis_name="c", subcore_axis_name="s")

    @functools.partial(
        pl.kernel,
        mesh=mesh,
        out_type=jax.ShapeDtypeStruct((N, D), jnp.float32),
        compiler_params=pltpu.CompilerParams(use_tc_tiling_on_sc=False),
        scratch_types=[
            pltpu.VMEM((CHUNK,), jnp.float32),
            pltpu.VMEM((NSUB, SUB), jnp.int32),
            pltpu.VMEM((CHUNK, D), jnp.float32),
            pltpu.SemaphoreType.DMA,
        ],
    )
    def k(x_hbm, ptab_hbm, out_hbm, xin_v, idx_v, g_v, sem):
        wid = lax.axis_index("s") * NC + lax.axis_index("c")
        base_w = wid * ROWS_W
        starts = [min(i * L, SUB - L) for i in range(SUB // L + 1)]
        offs = [
            1 + CARD * lax.rem(s + lax.iota(jnp.int32, L), F) for s in starts
        ]

        def body(c, carry):
            base = base_w + c * CHUNK
            pltpu.sync_copy(x_hbm.at[pl.ds(base, CHUNK)], xin_v)
            for j in range(NSUB):
                for i, s in enumerate(starts):
                    q = j * SUB + s
                    xi = xin_v[pl.ds(q, L)].astype(jnp.int32)
                    idx_v[j, pl.ds(s, L)] = xi + offs[i]
            copies = [
                pltpu.async_copy(
                    ptab_hbm.at[idx_v.at[j]],
                    g_v.at[pl.ds(j * SUB, SUB)],
                    sem,
                )
                for j in range(NSUB)
            ]
            for cp in copies:
                cp.wait()
            pltpu.sync_copy(g_v, out_hbm.at[pl.ds(base, CHUNK)])
            return carry

        lax.fori_loop(0, NCHUNK, body, 0)

    return k


_build_table = _build_table_kernel()
_gather = _gather_phase_kernel()


def kernel(inputs, feature_embedding, shared_embedding):
    x_flat = inputs.reshape(N)
    feat_flat = feature_embedding.reshape(V * D_FEAT)
    sh_pad = jnp.zeros((F * D_SH + L,), jnp.float32)
    sh_pad = sh_pad.at[: F * D_SH].set(shared_embedding.reshape(F * D_SH))
    ptab_flat = _build_table(feat_flat, sh_pad)
    ptab = ptab_flat.reshape(V, D)
    out = _gather(x_flat, ptab)
    return out.reshape(B, F, D)


# speedup vs baseline: 1.3980x; 1.3980x over previous
import functools

import jax
import jax.numpy as jnp
from jax import lax
from jax.experimental import pallas as pl
from jax.experimental.pallas import tpu as pltpu
from jax.experimental.pallas import tpu_sc as plsc

B = 16384
F = 100
CARD = 10000
D_FEAT = 28
D_SH = 4
D = 32
V = F * CARD + 1
NC, NS, L = 2, 16, 16
NW = NC * NS

BC = 2000
NBCH = (V - 1) // BC
BSLOT = (NBCH + NW - 1) // NW

C = 32
BPW = B // NW
NCH = BPW // C
FG = 4
NU = F // FG

_CP = pltpu.CompilerParams(
    use_tc_tiling_on_sc=False, needs_layout_passes=False
)


def _build_table_kernel():
    mesh = plsc.VectorSubcoreMesh(core_axis_name="c", subcore_axis_name="s")

    @functools.partial(
        pl.kernel,
        mesh=mesh,
        out_type=jax.ShapeDtypeStruct((V * D,), jnp.float32),
        compiler_params=_CP,
        scratch_types=[
            pltpu.VMEM((D_FEAT, BC), jnp.float32),
            pltpu.VMEM((BC * D,), jnp.float32),
            pltpu.VMEM((F * D_SH + L,), jnp.float32),
            pltpu.VMEM((D_FEAT, 1), jnp.float32),
            pltpu.SemaphoreType.DMA,
        ],
    )
    def k(featT_hbm, sh_hbm, ptab_hbm, fv, buf_v, sh_v, ftl, sem):
        wid = lax.axis_index("s") * NC + lax.axis_index("c")
        pltpu.sync_copy(sh_hbm, sh_v)
        iota = lax.iota(jnp.int32, L)
        lane4 = lax.rem(iota, 4)
        s32 = D * iota
        phead = D * lax.div(iota, 4) + lane4

        def do_chunk(c):
            t0 = c * BC
            pltpu.sync_copy(featT_hbm.at[:, pl.ds(t0, BC)], fv)
            f_main = t0 // CARD
            hrep = plsc.load_gather(sh_v, [D_SH * f_main + lane4])

            def j16(j0, carry):
                tj = j0 * L
                for d in range(D_FEAT):
                    v = fv[d, pl.ds(tj, L)]
                    plsc.store_scatter(buf_v, [s32 + (tj * D + D_SH + d)], v)
                for kk in range(4):
                    plsc.store_scatter(
                        buf_v, [phead + (tj + 4 * kk) * D], hrep
                    )
                return carry

            lax.fori_loop(0, BC // L, j16, 0)
            f0 = jnp.maximum(t0 - 1, 0) // CARD
            h0 = plsc.load_gather(sh_v, [D_SH * f0 + lane4])
            plsc.store_scatter(buf_v, [lane4], h0, mask=iota < D_SH)
            pltpu.sync_copy(buf_v, ptab_hbm.at[pl.ds(t0 * D, BC * D)])

        def slot(i, carry):
            c = wid + NW * i

            @pl.when(c < NBCH)
            def _():
                do_chunk(c)

            return carry

        lax.fori_loop(0, BSLOT, slot, 0)

        @pl.when(wid == NW - 1)
        def _():
            t = V - 1
            pltpu.sync_copy(featT_hbm.at[:, pl.ds(t, 1)], ftl)
            zero = jnp.zeros((L,), jnp.int32)
            v1 = plsc.load_gather(ftl, [iota, zero])
            v2 = plsc.load_gather(ftl, [iota + (D_FEAT - L), zero])
            buf_v[pl.ds(D_SH, L)] = v1
            buf_v[pl.ds(D_SH + D_FEAT - L, L)] = v2
            h99 = plsc.load_gather(sh_v, [D_SH * (F - 1) + lane4])
            plsc.store_scatter(buf_v, [lane4], h99, mask=iota < D_SH)
            pltpu.sync_copy(buf_v.at[pl.ds(0, D)], ptab_hbm.at[pl.ds(t * D, D)])

    return k


def _gather_phase_kernel():
    mesh = plsc.VectorSubcoreMesh(core_axis_name="c", subcore_axis_name="s")

    @functools.partial(
        pl.kernel,
        mesh=mesh,
        out_type=jax.ShapeDtypeStruct((F, D, B), jnp.float32),
        compiler_params=_CP,
        scratch_types=[
            pltpu.VMEM((F, C), jnp.float32),
            pltpu.VMEM((1, FG * C), jnp.int32),
            pltpu.VMEM((FG * C, D), jnp.float32),
            pltpu.VMEM((FG, D, C), jnp.float32),
            pltpu.SemaphoreType.DMA,
        ],
    )
    def k(xT_hbm, ptab_hbm, out_hbm, xv, idx_v, g_v, slab, sem):
        wid = lax.axis_index("s") * NC + lax.axis_index("c")
        b_w = wid * BPW
        iota = lax.iota(jnp.int32, L)
        zero = jnp.zeros((L,), jnp.int32)

        def chunk(ci, carry):
            b0 = b_w + ci * C
            pltpu.sync_copy(xT_hbm.at[:, pl.ds(b0, C)], xv)

            def sub(u, carry2):
                f0 = FG * u
                for m in range(2 * FG):
                    f = f0 + m // 2
                    xi = xv[f, pl.ds(L * (m % 2), L)].astype(jnp.int32)
                    idx_v[0, pl.ds(L * m, L)] = xi + (1 + CARD * f)
                pltpu.async_copy(
                    ptab_hbm.at[idx_v.at[0]], g_v, sem
                ).wait()

                def dloop(d, carry3):
                    dcol = zero + d
                    for fl in range(FG):
                        r1 = iota + fl * C
                        slab[fl, d, pl.ds(0, L)] = plsc.load_gather(
                            g_v, [r1, dcol]
                        )
                        slab[fl, d, pl.ds(L, L)] = plsc.load_gather(
                            g_v, [r1 + L, dcol]
                        )
                    return carry3

                lax.fori_loop(0, D, dloop, 0)
                pltpu.sync_copy(
                    slab, out_hbm.at[pl.ds(f0, FG), :, pl.ds(b0, C)]
                )
                return carry2

            lax.fori_loop(0, NU, sub, 0)
            return carry

        lax.fori_loop(0, NCH, chunk, 0)

    return k


_build_table = _build_table_kernel()
_gather = _gather_phase_kernel()


def kernel(inputs, feature_embedding, shared_embedding):
    xT = jnp.transpose(inputs)
    featT = jnp.transpose(feature_embedding)
    sh_pad = jnp.zeros((F * D_SH + L,), jnp.float32)
    sh_pad = sh_pad.at[: F * D_SH].set(shared_embedding.reshape(F * D_SH))
    ptab = _build_table(featT, sh_pad).reshape(V, D)
    outT = _gather(xT, ptab)
    return jnp.transpose(outT, (2, 0, 1))


# speedup vs baseline: 2.3569x; 1.6860x over previous
import functools

import jax
import jax.numpy as jnp
from jax import lax
from jax.experimental import pallas as pl
from jax.experimental.pallas import tpu as pltpu
from jax.experimental.pallas import tpu_sc as plsc

B = 16384
F = 100
CARD = 10000
D_FEAT = 28
D_SH = 4
D = 32
V = F * CARD + 1
NC, NS, L = 2, 16, 16
NW = NC * NS

BCHUNK = 1600
NBCHUNK = (V - 1) // BCHUNK
BITER = (NBCHUNK + NW - 1) // NW
SB = 400

C = 32
BPW = B // NW
NCH = BPW // C
FG = 4
NU = F // FG

_CP = pltpu.CompilerParams(
    use_tc_tiling_on_sc=False, needs_layout_passes=False
)


def _build_table_kernel():
    mesh = plsc.VectorSubcoreMesh(core_axis_name="c", subcore_axis_name="s")

    @functools.partial(
        pl.kernel,
        mesh=mesh,
        out_type=jax.ShapeDtypeStruct((V * D,), jnp.float32),
        compiler_params=_CP,
        scratch_types=[
            pltpu.VMEM((BCHUNK * D_FEAT,), jnp.float32),
            pltpu.VMEM((BCHUNK * D,), jnp.float32),
            pltpu.VMEM((F * D_SH + L,), jnp.float32),
            pltpu.SemaphoreType.DMA,
        ],
    )
    def k(feat_hbm, sh_hbm, ptab_hbm, fin_v, buf_v, sh_v, sem):
        wid = lax.axis_index("s") * NC + lax.axis_index("c")
        pltpu.sync_copy(sh_hbm, sh_v)

        def do_chunk(c):
            base = c * BCHUNK
            pltpu.sync_copy(
                feat_hbm.at[pl.ds(base * D_FEAT, BCHUNK * D_FEAT)], fin_v
            )
            for sb in range(BCHUNK // SB):
                t0 = base + sb * SB
                f_rest = t0 // CARD
                h = sh_v[pl.ds(D_SH * f_rest, L)]

                def row4(i, carry):
                    r = sb * SB + i * 4
                    for u in range(4):
                        o = (r + u) * D
                        g = (r + u) * D_FEAT
                        buf_v[pl.ds(o, L)] = h
                        buf_v[pl.ds(o + D_SH, L)] = fin_v[pl.ds(g, L)]
                        buf_v[pl.ds(o + D_SH + D_FEAT - L, L)] = fin_v[
                            pl.ds(g + D_FEAT - L, L)
                        ]
                    return carry

                lax.fori_loop(0, SB // 4, row4, 0)
                f_first = jnp.maximum(t0 - 1, 0) // CARD
                h0 = sh_v[pl.ds(D_SH * f_first, L)]
                buf_v[pl.ds(sb * SB * D, L)] = h0
                buf_v[pl.ds(sb * SB * D + D_SH, L)] = fin_v[
                    pl.ds(sb * SB * D_FEAT, L)
                ]
            pltpu.sync_copy(buf_v, ptab_hbm.at[pl.ds(base * D, BCHUNK * D)])

        def body(i, carry):
            c = wid + NW * i

            @pl.when(c < NBCHUNK)
            def _():
                do_chunk(c)

            return carry

        lax.fori_loop(0, BITER, body, 0)

        @pl.when(wid == NW - 1)
        def _():
            t = V - 1
            pltpu.sync_copy(
                feat_hbm.at[pl.ds(t * D_FEAT, D_FEAT)],
                fin_v.at[pl.ds(0, D_FEAT)],
            )
            buf_v[pl.ds(0, L)] = sh_v[pl.ds(D_SH * (F - 1), L)]
            buf_v[pl.ds(D_SH, L)] = fin_v[pl.ds(0, L)]
            buf_v[pl.ds(D_SH + D_FEAT - L, L)] = fin_v[pl.ds(D_FEAT - L, L)]
            pltpu.sync_copy(buf_v.at[pl.ds(0, D)], ptab_hbm.at[pl.ds(t * D, D)])

    return k


def _gather_phase_kernel():
    mesh = plsc.VectorSubcoreMesh(core_axis_name="c", subcore_axis_name="s")

    @functools.partial(
        pl.kernel,
        mesh=mesh,
        out_type=jax.ShapeDtypeStruct((F, D, B), jnp.float32),
        compiler_params=_CP,
        scratch_types=[
            pltpu.VMEM((F, C), jnp.float32),
            pltpu.VMEM((1, FG * C), jnp.int32),
            pltpu.VMEM((FG * C, D), jnp.float32),
            pltpu.VMEM((FG, D, C), jnp.float32),
            pltpu.SemaphoreType.DMA,
        ],
    )
    def k(xT_hbm, ptab_hbm, out_hbm, xv, idx_v, g_v, slab, sem):
        wid = lax.axis_index("s") * NC + lax.axis_index("c")
        b_w = wid * BPW
        iota = lax.iota(jnp.int32, L)
        rows = [iota + fl * C for fl in range(FG)] + [
            iota + (fl * C + L) for fl in range(FG)
        ]

        def chunk(ci, carry):
            b0 = b_w + ci * C
            pltpu.sync_copy(xT_hbm.at[:, pl.ds(b0, C)], xv)

            def sub(u, carry2):
                f0 = FG * u
                for m in range(2 * FG):
                    f = f0 + m // 2
                    xi = xv[f, pl.ds(L * (m % 2), L)].astype(jnp.int32)
                    idx_v[0, pl.ds(L * m, L)] = xi + (1 + CARD * f)
                pltpu.async_copy(ptab_hbm.at[idx_v.at[0]], g_v, sem).wait()
                for d in range(D):
                    dcol = jnp.full((L,), d, jnp.int32)
                    for fl in range(FG):
                        slab[fl, d, pl.ds(0, L)] = plsc.load_gather(
                            g_v, [rows[fl], dcol]
                        )
                        slab[fl, d, pl.ds(L, L)] = plsc.load_gather(
                            g_v, [rows[FG + fl], dcol]
                        )
                pltpu.sync_copy(
                    slab, out_hbm.at[pl.ds(f0, FG), :, pl.ds(b0, C)]
                )
                return carry2

            lax.fori_loop(0, NU, sub, 0)
            return carry

        lax.fori_loop(0, NCH, chunk, 0)

    return k


_build_table = _build_table_kernel()
_gather = _gather_phase_kernel()


def kernel(inputs, feature_embedding, shared_embedding):
    xT = jnp.transpose(inputs)
    feat_flat = feature_embedding.reshape(V * D_FEAT)
    sh_pad = jnp.zeros((F * D_SH + L,), jnp.float32)
    sh_pad = sh_pad.at[: F * D_SH].set(shared_embedding.reshape(F * D_SH))
    ptab = _build_table(feat_flat, sh_pad).reshape(V, D)
    outT = _gather(xT, ptab)
    return jnp.transpose(outT, (2, 0, 1))


# speedup vs baseline: 2.5110x; 1.0654x over previous
import functools

import jax
import jax.numpy as jnp
from jax import lax
from jax.experimental import pallas as pl
from jax.experimental.pallas import tpu as pltpu
from jax.experimental.pallas import tpu_sc as plsc

B = 16384
F = 100
CARD = 10000
D_FEAT = 28
D_SH = 4
D = 32
V = F * CARD + 1
NC, NS, L = 2, 16, 16
NW = NC * NS

BCHUNK = 1600
NBCHUNK = (V - 1) // BCHUNK
BITER = (NBCHUNK + NW - 1) // NW
SB = 400

C = 32
BPW = B // NW
NCH = BPW // C
FG = 4
NU = F // FG
R = 4

_CP = pltpu.CompilerParams(
    use_tc_tiling_on_sc=False, needs_layout_passes=False
)


def _build_table_kernel():
    mesh = plsc.VectorSubcoreMesh(core_axis_name="c", subcore_axis_name="s")

    @functools.partial(
        pl.kernel,
        mesh=mesh,
        out_type=jax.ShapeDtypeStruct((V * D,), jnp.float32),
        compiler_params=_CP,
        scratch_types=[
            pltpu.VMEM((BCHUNK, D_FEAT), jnp.float32),
            pltpu.VMEM((BCHUNK * D,), jnp.float32),
            pltpu.VMEM((F * D_SH + L,), jnp.float32),
            pltpu.SemaphoreType.DMA,
        ],
    )
    def k(feat_hbm, sh_hbm, ptab_hbm, fin_v, buf_v, sh_v, sem):
        wid = lax.axis_index("s") * NC + lax.axis_index("c")
        pltpu.sync_copy(sh_hbm, sh_v)

        def do_chunk(c):
            base = c * BCHUNK
            pltpu.sync_copy(feat_hbm.at[pl.ds(base, BCHUNK), :], fin_v)
            for sb in range(BCHUNK // SB):
                t0 = base + sb * SB
                f_rest = t0 // CARD
                h = sh_v[pl.ds(D_SH * f_rest, L)]

                def row4(i, carry):
                    r = sb * SB + i * 4
                    for u in range(4):
                        o = (r + u) * D
                        buf_v[pl.ds(o, L)] = h
                        buf_v[pl.ds(o + D_SH, L)] = fin_v[r + u, pl.ds(0, L)]
                        buf_v[pl.ds(o + D_SH + D_FEAT - L, L)] = fin_v[
                            r + u, pl.ds(D_FEAT - L, L)
                        ]
                    return carry

                lax.fori_loop(0, SB // 4, row4, 0)
                f_first = jnp.maximum(t0 - 1, 0) // CARD
                h0 = sh_v[pl.ds(D_SH * f_first, L)]
                buf_v[pl.ds(sb * SB * D, L)] = h0
                buf_v[pl.ds(sb * SB * D + D_SH, L)] = fin_v[
                    sb * SB, pl.ds(0, L)
                ]
            pltpu.sync_copy(buf_v, ptab_hbm.at[pl.ds(base * D, BCHUNK * D)])

        def body(i, carry):
            c = wid + NW * i

            @pl.when(c < NBCHUNK)
            def _():
                do_chunk(c)

            return carry

        lax.fori_loop(0, BITER, body, 0)

        @pl.when(wid == NW - 1)
        def _():
            t = V - 1
            pltpu.sync_copy(feat_hbm.at[pl.ds(t, 1), :], fin_v.at[pl.ds(0, 1), :])
            buf_v[pl.ds(0, L)] = sh_v[pl.ds(D_SH * (F - 1), L)]
            buf_v[pl.ds(D_SH, L)] = fin_v[0, pl.ds(0, L)]
            buf_v[pl.ds(D_SH + D_FEAT - L, L)] = fin_v[0, pl.ds(D_FEAT - L, L)]
            pltpu.sync_copy(buf_v.at[pl.ds(0, D)], ptab_hbm.at[pl.ds(t * D, D)])

    return k


def _gather_phase_kernel():
    mesh = plsc.VectorSubcoreMesh(core_axis_name="c", subcore_axis_name="s")

    @functools.partial(
        pl.kernel,
        mesh=mesh,
        out_type=jax.ShapeDtypeStruct((F, D, B), jnp.float32),
        compiler_params=_CP,
        scratch_types=[
            pltpu.VMEM((F, C), jnp.float32),
            pltpu.VMEM((R, FG * C), jnp.int32),
            pltpu.VMEM((FG * C, D), jnp.float32),
            pltpu.VMEM((FG * C, D), jnp.float32),
            pltpu.VMEM((FG * C, D), jnp.float32),
            pltpu.VMEM((FG * C, D), jnp.float32),
            pltpu.VMEM((FG, D, C), jnp.float32),
            pltpu.VMEM((FG, D, C), jnp.float32),
            pltpu.SemaphoreType.DMA,
            pltpu.SemaphoreType.DMA,
            pltpu.SemaphoreType.DMA,
            pltpu.SemaphoreType.DMA,
            pltpu.SemaphoreType.DMA,
            pltpu.SemaphoreType.DMA,
        ],
    )
    def k(xT_hbm, ptab_hbm, out_hbm, xv, idx_v,
          g0, g1, g2, g3, sl0, sl1, q0, q1, q2, q3, o0, o1):
        gs = [g0, g1, g2, g3]
        qs = [q0, q1, q2, q3]
        sls = [sl0, sl1]
        os_ = [o0, o1]
        wid = lax.axis_index("s") * NC + lax.axis_index("c")
        b_w = wid * BPW
        iota = lax.iota(jnp.int32, L)
        rows = [iota + fl * C for fl in range(FG)] + [
            iota + (fl * C + L) for fl in range(FG)
        ]

        def write_idx(slot, u):
            f0 = FG * u
            for m in range(2 * FG):
                f = f0 + m // 2
                xi = xv[f, pl.ds(L * (m % 2), L)].astype(jnp.int32)
                idx_v[slot, pl.ds(L * m, L)] = xi + (1 + CARD * f)

        def issue_gather(slot):
            pltpu.async_copy(ptab_hbm.at[idx_v.at[slot]], gs[slot], qs[slot])

        def wait_gather(slot):
            pltpu.make_async_copy(
                ptab_hbm.at[pl.ds(0, FG * C)], gs[slot], qs[slot]
            ).wait()

        def transpose(slot, s):
            for d in range(D):
                dcol = jnp.full((L,), d, jnp.int32)
                for fl in range(FG):
                    sls[s][fl, d, pl.ds(0, L)] = plsc.load_gather(
                        gs[slot], [rows[fl], dcol]
                    )
                    sls[s][fl, d, pl.ds(L, L)] = plsc.load_gather(
                        gs[slot], [rows[FG + fl], dcol]
                    )

        def issue_out(s, f0, b0):
            pltpu.async_copy(
                sls[s], out_hbm.at[pl.ds(f0, FG), :, pl.ds(b0, C)], os_[s]
            )

        def wait_out(s):
            pltpu.make_async_copy(
                sls[s], out_hbm.at[pl.ds(0, FG), :, pl.ds(0, C)], os_[s]
            ).wait()

        def chunk(ci, carry):
            b0 = b_w + ci * C
            pltpu.sync_copy(xT_hbm.at[:, pl.ds(b0, C)], xv)
            for j in range(R - 1):
                write_idx(j, j)
                issue_gather(j)

            def group(i, carry2):
                for j in range(R):
                    u = R * i + j
                    wait_gather(j)
                    unext = u + (R - 1)

                    @pl.when(unext < NU)
                    def _():
                        write_idx((j + R - 1) % R, unext)
                        issue_gather((j + R - 1) % R)

                    s = j % 2
                    if j < 2:
                        @pl.when(i > 0)
                        def _():
                            wait_out(s)
                    else:
                        wait_out(s)
                    transpose(j, s)
                    issue_out(s, FG * u, b0)
                return carry2

            lax.fori_loop(0, (NU - 1) // R, group, 0)
            wait_gather(0)
            wait_out(0)
            transpose(0, 0)
            issue_out(0, FG * (NU - 1), b0)
            wait_out(0)
            wait_out(1)
            return carry

        lax.fori_loop(0, NCH, chunk, 0)

    return k


_build_table = _build_table_kernel()
_gather = _gather_phase_kernel()


def kernel(inputs, feature_embedding, shared_embedding):
    xT = jnp.transpose(inputs)
    sh_pad = jnp.zeros((F * D_SH + L,), jnp.float32)
    sh_pad = sh_pad.at[: F * D_SH].set(shared_embedding.reshape(F * D_SH))
    ptab = _build_table(feature_embedding, sh_pad).reshape(V, D)
    outT = _gather(xT, ptab)
    return jnp.transpose(outT, (2, 0, 1))


# speedup vs baseline: 3.5277x; 1.4049x over previous
import functools

import jax
import jax.numpy as jnp
from jax import lax
from jax.experimental import pallas as pl
from jax.experimental.pallas import tpu as pltpu
from jax.experimental.pallas import tpu_sc as plsc

B = 16384
F = 100
CARD = 10000
D_FEAT = 28
D_SH = 4
D = 32
V = F * CARD + 1
NC, NS, L = 2, 16, 16
NW = NC * NS

BCHUNK = 1600
NBCHUNK = (V - 1) // BCHUNK
BITER = (NBCHUNK + NW - 1) // NW
SB = 400

C = 32
BPW = B // NW
NCH = BPW // C
FG = 4
NU = F // FG
R = 4

_CP = pltpu.CompilerParams(
    use_tc_tiling_on_sc=False, needs_layout_passes=False
)


def _build_table_kernel():
    mesh = plsc.VectorSubcoreMesh(core_axis_name="c", subcore_axis_name="s")

    @functools.partial(
        pl.kernel,
        mesh=mesh,
        out_type=jax.ShapeDtypeStruct((V * D,), jnp.float32),
        compiler_params=_CP,
        scratch_types=[
            pltpu.VMEM((BCHUNK, D_FEAT), jnp.float32),
            pltpu.VMEM((BCHUNK * D,), jnp.float32),
            pltpu.VMEM((F * D_SH + L,), jnp.float32),
            pltpu.SemaphoreType.DMA,
        ],
    )
    def k(feat_hbm, sh_hbm, ptab_hbm, fin_v, buf_v, sh_v, sem):
        wid = lax.axis_index("s") * NC + lax.axis_index("c")
        pltpu.sync_copy(sh_hbm, sh_v)

        def do_chunk(c):
            base = c * BCHUNK
            pltpu.sync_copy(feat_hbm.at[pl.ds(base, BCHUNK), :], fin_v)
            for sb in range(BCHUNK // SB):
                t0 = base + sb * SB
                f_rest = t0 // CARD
                h = sh_v[pl.ds(D_SH * f_rest, L)]

                def row4(i, carry):
                    r = sb * SB + i * 4
                    for u in range(4):
                        o = (r + u) * D
                        buf_v[pl.ds(o, L)] = h
                        buf_v[pl.ds(o + D_SH, L)] = fin_v[r + u, pl.ds(0, L)]
                        buf_v[pl.ds(o + D_SH + D_FEAT - L, L)] = fin_v[
                            r + u, pl.ds(D_FEAT - L, L)
                        ]
                    return carry

                lax.fori_loop(0, SB // 4, row4, 0)
                f_first = jnp.maximum(t0 - 1, 0) // CARD
                h0 = sh_v[pl.ds(D_SH * f_first, L)]
                buf_v[pl.ds(sb * SB * D, L)] = h0
                buf_v[pl.ds(sb * SB * D + D_SH, L)] = fin_v[
                    sb * SB, pl.ds(0, L)
                ]
            pltpu.sync_copy(buf_v, ptab_hbm.at[pl.ds(base * D, BCHUNK * D)])

        def body(i, carry):
            c = wid + NW * i

            @pl.when(c < NBCHUNK)
            def _():
                do_chunk(c)

            return carry

        lax.fori_loop(0, BITER, body, 0)

        @pl.when(wid == NW - 1)
        def _():
            t = V - 1
            pltpu.sync_copy(feat_hbm.at[pl.ds(t, 1), :], fin_v.at[pl.ds(0, 1), :])
            buf_v[pl.ds(0, L)] = sh_v[pl.ds(D_SH * (F - 1), L)]
            buf_v[pl.ds(D_SH, L)] = fin_v[0, pl.ds(0, L)]
            buf_v[pl.ds(D_SH + D_FEAT - L, L)] = fin_v[0, pl.ds(D_FEAT - L, L)]
            pltpu.sync_copy(buf_v.at[pl.ds(0, D)], ptab_hbm.at[pl.ds(t * D, D)])

    return k


def _gather_phase_kernel():
    mesh = plsc.VectorSubcoreMesh(core_axis_name="c", subcore_axis_name="s")

    @functools.partial(
        pl.kernel,
        mesh=mesh,
        out_type=jax.ShapeDtypeStruct((F, D, B), jnp.float32),
        compiler_params=_CP,
        scratch_types=[
            pltpu.VMEM((F, C), jnp.float32),
            pltpu.VMEM((R, FG * C), jnp.int32),
            pltpu.VMEM((FG * C, D), jnp.float32),
            pltpu.VMEM((FG * C, D), jnp.float32),
            pltpu.VMEM((FG * C, D), jnp.float32),
            pltpu.VMEM((FG * C, D), jnp.float32),
            pltpu.VMEM((FG, D, C + 1), jnp.float32),
            pltpu.VMEM((FG, D, C + 1), jnp.float32),
            pltpu.SemaphoreType.DMA,
            pltpu.SemaphoreType.DMA,
            pltpu.SemaphoreType.DMA,
            pltpu.SemaphoreType.DMA,
            pltpu.SemaphoreType.DMA,
            pltpu.SemaphoreType.DMA,
        ],
    )
    def k(xT_hbm, ptab_hbm, out_hbm, xv, idx_v,
          g0, g1, g2, g3, sl0, sl1, q0, q1, q2, q3, o0, o1):
        gs = [g0, g1, g2, g3]
        qs = [q0, q1, q2, q3]
        sls = [sl0, sl1]
        os_ = [o0, o1]
        wid = lax.axis_index("s") * NC + lax.axis_index("c")
        b_w = wid * BPW
        iota = lax.iota(jnp.int32, L)
        rows = [iota + fl * C for fl in range(FG)] + [
            iota + (fl * C + L) for fl in range(FG)
        ]

        def write_idx(slot, u):
            f0 = FG * u
            for m in range(2 * FG):
                f = f0 + m // 2
                xi = xv[f, pl.ds(L * (m % 2), L)].astype(jnp.int32)
                idx_v[slot, pl.ds(L * m, L)] = xi + (1 + CARD * f)

        def issue_gather(slot):
            pltpu.async_copy(ptab_hbm.at[idx_v.at[slot]], gs[slot], qs[slot])

        def wait_gather(slot):
            pltpu.make_async_copy(
                ptab_hbm.at[pl.ds(0, FG * C)], gs[slot], qs[slot]
            ).wait()

        dlo = lax.iota(jnp.int32, L)
        dhi = dlo + L

        def transpose(slot, s):
            for fl in range(FG):
                flv = jnp.full((L,), fl, jnp.int32)
                for jb in range(C):
                    bv = jnp.full((L,), jb, jnp.int32)
                    j = fl * C + jb
                    plsc.store_scatter(
                        sls[s], [flv, dlo, bv], gs[slot][j, pl.ds(0, L)]
                    )
                    plsc.store_scatter(
                        sls[s], [flv, dhi, bv], gs[slot][j, pl.ds(L, L)]
                    )

        def issue_out(s, f0, b0):
            pltpu.async_copy(
                sls[s].at[:, :, pl.ds(0, C)],
                out_hbm.at[pl.ds(f0, FG), :, pl.ds(b0, C)],
                os_[s],
            )

        def wait_out(s):
            pltpu.make_async_copy(
                sls[s].at[:, :, pl.ds(0, C)],
                out_hbm.at[pl.ds(0, FG), :, pl.ds(0, C)],
                os_[s],
            ).wait()

        def chunk(ci, carry):
            b0 = b_w + ci * C
            pltpu.sync_copy(xT_hbm.at[:, pl.ds(b0, C)], xv)
            for j in range(R - 1):
                write_idx(j, j)
                issue_gather(j)

            def group(i, carry2):
                for j in range(R):
                    u = R * i + j
                    wait_gather(j)
                    unext = u + (R - 1)

                    @pl.when(unext < NU)
                    def _():
                        write_idx((j + R - 1) % R, unext)
                        issue_gather((j + R - 1) % R)

                    s = j % 2
                    if j < 2:
                        @pl.when(i > 0)
                        def _():
                            wait_out(s)
                    else:
                        wait_out(s)
                    transpose(j, s)
                    issue_out(s, FG * u, b0)
                return carry2

            lax.fori_loop(0, (NU - 1) // R, group, 0)
            wait_gather(0)
            wait_out(0)
            transpose(0, 0)
            issue_out(0, FG * (NU - 1), b0)
            wait_out(0)
            wait_out(1)
            return carry

        lax.fori_loop(0, NCH, chunk, 0)

    return k


_build_table = _build_table_kernel()
_gather = _gather_phase_kernel()


def kernel(inputs, feature_embedding, shared_embedding):
    xT = jnp.transpose(inputs)
    sh_pad = jnp.zeros((F * D_SH + L,), jnp.float32)
    sh_pad = sh_pad.at[: F * D_SH].set(shared_embedding.reshape(F * D_SH))
    ptab = _build_table(feature_embedding, sh_pad).reshape(V, D)
    outT = _gather(xT, ptab)
    return jnp.transpose(outT, (2, 0, 1))


# speedup vs baseline: 4.1406x; 1.1737x over previous
import functools

import jax
import jax.numpy as jnp
from jax import lax
from jax.experimental import pallas as pl
from jax.experimental.pallas import tpu as pltpu
from jax.experimental.pallas import tpu_sc as plsc

B = 16384
F = 100
CARD = 10000
D_FEAT = 28
D_SH = 4
D = 32
V = F * CARD + 1
NC, NS, L = 2, 16, 16
NW = NC * NS

BCHUNK = 1600
NBCHUNK = (V - 1) // BCHUNK
BITER = (NBCHUNK + NW - 1) // NW
SB = 400

C = 32
BPW = B // NW
NCH = BPW // C
FG = 4
NU = F // FG
R = 4

_CP = pltpu.CompilerParams(
    use_tc_tiling_on_sc=False, needs_layout_passes=False
)


def _build_table_kernel():
    mesh = plsc.VectorSubcoreMesh(core_axis_name="c", subcore_axis_name="s")

    @functools.partial(
        pl.kernel,
        mesh=mesh,
        out_type=jax.ShapeDtypeStruct((V * D,), jnp.float32),
        compiler_params=_CP,
        scratch_types=[
            pltpu.VMEM((BCHUNK, D_FEAT), jnp.float32),
            pltpu.VMEM((BCHUNK * D,), jnp.float32),
            pltpu.VMEM((F * D_SH + L,), jnp.float32),
            pltpu.SemaphoreType.DMA,
        ],
    )
    def k(feat_hbm, sh_hbm, ptab_hbm, fin_v, buf_v, sh_v, sem):
        wid = lax.axis_index("s") * NC + lax.axis_index("c")
        pltpu.sync_copy(sh_hbm, sh_v)

        def do_chunk(c):
            base = c * BCHUNK
            pltpu.sync_copy(feat_hbm.at[pl.ds(base, BCHUNK), :], fin_v)
            for sb in range(BCHUNK // SB):
                t0 = base + sb * SB
                f_rest = t0 // CARD
                h = sh_v[pl.ds(D_SH * f_rest, L)]

                def row4(i, carry):
                    r = sb * SB + i * 4
                    for u in range(4):
                        o = (r + u) * D
                        buf_v[pl.ds(o, L)] = h
                        buf_v[pl.ds(o + D_SH, L)] = fin_v[r + u, pl.ds(0, L)]
                        buf_v[pl.ds(o + D_SH + D_FEAT - L, L)] = fin_v[
                            r + u, pl.ds(D_FEAT - L, L)
                        ]
                    return carry

                lax.fori_loop(0, SB // 4, row4, 0)
                f_first = jnp.maximum(t0 - 1, 0) // CARD
                h0 = sh_v[pl.ds(D_SH * f_first, L)]
                buf_v[pl.ds(sb * SB * D, L)] = h0
                buf_v[pl.ds(sb * SB * D + D_SH, L)] = fin_v[
                    sb * SB, pl.ds(0, L)
                ]
            pltpu.sync_copy(buf_v, ptab_hbm.at[pl.ds(base * D, BCHUNK * D)])

        def body(i, carry):
            c = wid + NW * i

            @pl.when(c < NBCHUNK)
            def _():
                do_chunk(c)

            return carry

        lax.fori_loop(0, BITER, body, 0)

        @pl.when(wid == NW - 1)
        def _():
            t = V - 1
            pltpu.sync_copy(feat_hbm.at[pl.ds(t, 1), :], fin_v.at[pl.ds(0, 1), :])
            buf_v[pl.ds(0, L)] = sh_v[pl.ds(D_SH * (F - 1), L)]
            buf_v[pl.ds(D_SH, L)] = fin_v[0, pl.ds(0, L)]
            buf_v[pl.ds(D_SH + D_FEAT - L, L)] = fin_v[0, pl.ds(D_FEAT - L, L)]
            pltpu.sync_copy(buf_v.at[pl.ds(0, D)], ptab_hbm.at[pl.ds(t * D, D)])

    return k


def _gather_phase_kernel():
    mesh = plsc.VectorSubcoreMesh(core_axis_name="c", subcore_axis_name="s")

    @functools.partial(
        pl.kernel,
        mesh=mesh,
        out_type=jax.ShapeDtypeStruct(
            (F, D // 8, B // 128, 8, 128), jnp.float32
        ),
        compiler_params=_CP,
        scratch_types=[
            pltpu.VMEM((F, C), jnp.float32),
            pltpu.VMEM((R, FG * C), jnp.int32),
            pltpu.VMEM((FG * C, D), jnp.float32),
            pltpu.VMEM((FG * C, D), jnp.float32),
            pltpu.VMEM((FG * C, D), jnp.float32),
            pltpu.VMEM((FG * C, D), jnp.float32),
            pltpu.VMEM((FG, D // 8, 8, C + 1), jnp.float32),
            pltpu.VMEM((FG, D // 8, 8, C + 1), jnp.float32),
            pltpu.SemaphoreType.DMA,
            pltpu.SemaphoreType.DMA,
            pltpu.SemaphoreType.DMA,
            pltpu.SemaphoreType.DMA,
            pltpu.SemaphoreType.DMA,
            pltpu.SemaphoreType.DMA,
        ],
    )
    def k(xT_hbm, ptab_hbm, out_hbm, xv, idx_v,
          g0, g1, g2, g3, sl0, sl1, q0, q1, q2, q3, o0, o1):
        gs = [g0, g1, g2, g3]
        qs = [q0, q1, q2, q3]
        sls = [sl0, sl1]
        os_ = [o0, o1]
        wid = lax.axis_index("s") * NC + lax.axis_index("c")
        b_w = wid * BPW
        iota = lax.iota(jnp.int32, L)
        rows = [iota + fl * C for fl in range(FG)] + [
            iota + (fl * C + L) for fl in range(FG)
        ]

        def write_idx(slot, u):
            f0 = FG * u
            for m in range(2 * FG):
                f = f0 + m // 2
                xi = xv[f, pl.ds(L * (m % 2), L)].astype(jnp.int32)
                idx_v[slot, pl.ds(L * m, L)] = xi + (1 + CARD * f)

        def issue_gather(slot):
            pltpu.async_copy(ptab_hbm.at[idx_v.at[slot]], gs[slot], qs[slot])

        def wait_gather(slot):
            pltpu.make_async_copy(
                ptab_hbm.at[pl.ds(0, FG * C)], gs[slot], qs[slot]
            ).wait()

        dvec = lax.iota(jnp.int32, L)
        dqlo, drlo = lax.div(dvec, 8), lax.rem(dvec, 8)
        dqhi, drhi = lax.div(dvec + L, 8), lax.rem(dvec + L, 8)

        def transpose(slot, s):
            for fl in range(FG):
                flv = jnp.full((L,), fl, jnp.int32)
                for jb in range(C):
                    bv = jnp.full((L,), jb, jnp.int32)
                    j = fl * C + jb
                    plsc.store_scatter(
                        sls[s], [flv, dqlo, drlo, bv], gs[slot][j, pl.ds(0, L)]
                    )
                    plsc.store_scatter(
                        sls[s], [flv, dqhi, drhi, bv], gs[slot][j, pl.ds(L, L)]
                    )

        def issue_out(s, f0, b0):
            pltpu.async_copy(
                sls[s].at[:, :, :, pl.ds(0, C)],
                out_hbm.at[
                    pl.ds(f0, FG), :, lax.div(b0, 128), :,
                    pl.ds(lax.rem(b0, 128), C),
                ],
                os_[s],
            )

        def wait_out(s):
            pltpu.make_async_copy(
                sls[s].at[:, :, :, pl.ds(0, C)],
                out_hbm.at[pl.ds(0, FG), :, 0, :, pl.ds(0, C)],
                os_[s],
            ).wait()

        def chunk(ci, carry):
            b0 = b_w + ci * C
            pltpu.sync_copy(xT_hbm.at[:, pl.ds(b0, C)], xv)
            for j in range(R - 1):
                write_idx(j, j)
                issue_gather(j)

            def group(i, carry2):
                for j in range(R):
                    u = R * i + j
                    wait_gather(j)
                    unext = u + (R - 1)

                    @pl.when(unext < NU)
                    def _():
                        write_idx((j + R - 1) % R, unext)
                        issue_gather((j + R - 1) % R)

                    s = j % 2
                    if j < 2:
                        @pl.when(i > 0)
                        def _():
                            wait_out(s)
                    else:
                        wait_out(s)
                    transpose(j, s)
                    issue_out(s, FG * u, b0)
                return carry2

            lax.fori_loop(0, (NU - 1) // R, group, 0)
            wait_gather(0)
            wait_out(0)
            transpose(0, 0)
            issue_out(0, FG * (NU - 1), b0)
            wait_out(0)
            wait_out(1)
            return carry

        lax.fori_loop(0, NCH, chunk, 0)

    return k


_build_table = _build_table_kernel()
_gather = _gather_phase_kernel()


def kernel(inputs, feature_embedding, shared_embedding):
    xT = jnp.transpose(inputs)
    sh_pad = jnp.zeros((F * D_SH + L,), jnp.float32)
    sh_pad = sh_pad.at[: F * D_SH].set(shared_embedding.reshape(F * D_SH))
    ptab = _build_table(feature_embedding, sh_pad).reshape(V, D)
    out5 = _gather(xT, ptab)
    outT = jnp.transpose(out5, (0, 1, 3, 2, 4)).reshape(F, D, B)
    return jnp.transpose(outT, (2, 0, 1))


# speedup vs baseline: 5.7245x; 1.3825x over previous
import functools

import jax
import jax.numpy as jnp
from jax import lax
from jax.experimental import pallas as pl
from jax.experimental.pallas import tpu as pltpu
from jax.experimental.pallas import tpu_sc as plsc

B = 16384
F = 100
CARD = 10000
D_FEAT = 28
D_SH = 4
D = 32
V = F * CARD + 1
NC, NS, L = 2, 16, 16
NW = NC * NS

VP = 1000064
TB = VP // 128
NBC = 13
BCT = NBC * 128
NCHB = (V + BCT - 1) // BCT
BSLOT4 = (NCHB + NW - 1) // NW
LASTN = V - (NCHB - 1) * BCT

C = 32
BPW = B // NW
NCH = BPW // C
FG = 4
NU = F // FG
R = 4

_CP = pltpu.CompilerParams(
    use_tc_tiling_on_sc=False, needs_layout_passes=False
)


def _build_table_kernel():
    mesh = plsc.VectorSubcoreMesh(core_axis_name="c", subcore_axis_name="s")

    @functools.partial(
        pl.kernel,
        mesh=mesh,
        out_type=jax.ShapeDtypeStruct((V, D), jnp.float32),
        compiler_params=_CP,
        scratch_types=[
            pltpu.VMEM((4, NBC, 8, 128), jnp.float32),
            pltpu.VMEM((BCT, D + 1), jnp.float32),
            pltpu.VMEM((F * D_SH + L * 2,), jnp.float32),
            pltpu.SemaphoreType.DMA,
        ],
    )
    def k(feat5_hbm, sh_hbm, ptab_hbm, ftv, buf_v, sh_v, sem):
        wid = lax.axis_index("s") * NC + lax.axis_index("c")
        pltpu.sync_copy(sh_hbm, sh_v)
        iota = lax.iota(jnp.int32, L)
        lane4 = lax.rem(iota, 4)
        hrow = lax.div(iota, 4)

        def do_chunk(c):
            t0 = c * BCT
            pltpu.sync_copy(feat5_hbm.at[:, pl.ds(c * NBC, NBC), :, :], ftv)

            def blk(jb, carry):
                tjb = t0 + jb * 128
                rjb = jb * 128
                for hg in range(32):
                    tvec = (tjb + 4 * hg) + hrow
                    fvec = lax.div(jnp.maximum(tvec - 1, 0), CARD)
                    h = plsc.load_gather(sh_v, [D_SH * fvec + lane4])
                    rvec = (rjb + 4 * hg) + hrow
                    plsc.store_scatter(buf_v, [rvec, lane4], h)
                for dq in range(4):
                    for dr in range(8):
                        d = dq * 8 + dr
                        if d >= D_FEAT:
                            continue
                        cvec = jnp.full((L,), D_SH + d, jnp.int32)
                        for hh in range(8):
                            v = ftv[dq, jb, dr, pl.ds(16 * hh, L)]
                            rv = (rjb + 16 * hh) + iota
                            plsc.store_scatter(buf_v, [rv, cvec], v)
                return carry

            lax.fori_loop(0, NBC, blk, 0)

            @pl.when(c < NCHB - 1)
            def _():
                pltpu.sync_copy(
                    buf_v.at[:, pl.ds(0, D)], ptab_hbm.at[pl.ds(t0, BCT), :]
                )

            @pl.when(c == NCHB - 1)
            def _():
                pltpu.sync_copy(
                    buf_v.at[pl.ds(0, LASTN), pl.ds(0, D)],
                    ptab_hbm.at[pl.ds(t0, LASTN), :],
                )

        def slot(i, carry):
            c = wid + NW * i

            @pl.when(c < NCHB)
            def _():
                do_chunk(c)

            return carry

        lax.fori_loop(0, BSLOT4, slot, 0)

    return k


def _gather_phase_kernel():
    mesh = plsc.VectorSubcoreMesh(core_axis_name="c", subcore_axis_name="s")

    @functools.partial(
        pl.kernel,
        mesh=mesh,
        out_type=jax.ShapeDtypeStruct(
            (F, D // 8, B // 128, 8, 128), jnp.float32
        ),
        compiler_params=_CP,
        scratch_types=[
            pltpu.VMEM((F, C), jnp.float32),
            pltpu.VMEM((R, FG * C), jnp.int32),
            pltpu.VMEM((FG * C, D), jnp.float32),
            pltpu.VMEM((FG * C, D), jnp.float32),
            pltpu.VMEM((FG * C, D), jnp.float32),
            pltpu.VMEM((FG * C, D), jnp.float32),
            pltpu.VMEM((FG, D // 8, 8, C + 1), jnp.float32),
            pltpu.VMEM((FG, D // 8, 8, C + 1), jnp.float32),
            pltpu.SemaphoreType.DMA,
            pltpu.SemaphoreType.DMA,
            pltpu.SemaphoreType.DMA,
            pltpu.SemaphoreType.DMA,
            pltpu.SemaphoreType.DMA,
            pltpu.SemaphoreType.DMA,
        ],
    )
    def k(xT_hbm, ptab_hbm, out_hbm, xv, idx_v,
          g0, g1, g2, g3, sl0, sl1, q0, q1, q2, q3, o0, o1):
        gs = [g0, g1, g2, g3]
        qs = [q0, q1, q2, q3]
        sls = [sl0, sl1]
        os_ = [o0, o1]
        wid = lax.axis_index("s") * NC + lax.axis_index("c")
        b_w = wid * BPW
        iota = lax.iota(jnp.int32, L)
        rows = [iota + fl * C for fl in range(FG)] + [
            iota + (fl * C + L) for fl in range(FG)
        ]

        def write_idx(slot, u):
            f0 = FG * u
            for m in range(2 * FG):
                f = f0 + m // 2
                xi = xv[f, pl.ds(L * (m % 2), L)].astype(jnp.int32)
                idx_v[slot, pl.ds(L * m, L)] = xi + (1 + CARD * f)

        def issue_gather(slot):
            pltpu.async_copy(ptab_hbm.at[idx_v.at[slot]], gs[slot], qs[slot])

        def wait_gather(slot):
            pltpu.make_async_copy(
                ptab_hbm.at[pl.ds(0, FG * C)], gs[slot], qs[slot]
            ).wait()

        dvec = lax.iota(jnp.int32, L)
        dqlo, drlo = lax.div(dvec, 8), lax.rem(dvec, 8)
        dqhi, drhi = lax.div(dvec + L, 8), lax.rem(dvec + L, 8)

        def transpose(slot, s):
            for fl in range(FG):
                flv = jnp.full((L,), fl, jnp.int32)
                for jb in range(C):
                    bv = jnp.full((L,), jb, jnp.int32)
                    j = fl * C + jb
                    plsc.store_scatter(
                        sls[s], [flv, dqlo, drlo, bv], gs[slot][j, pl.ds(0, L)]
                    )
                    plsc.store_scatter(
                        sls[s], [flv, dqhi, drhi, bv], gs[slot][j, pl.ds(L, L)]
                    )

        def issue_out(s, f0, b0):
            pltpu.async_copy(
                sls[s].at[:, :, :, pl.ds(0, C)],
                out_hbm.at[
                    pl.ds(f0, FG), :, lax.div(b0, 128), :,
                    pl.ds(lax.rem(b0, 128), C),
                ],
                os_[s],
            )

        def wait_out(s):
            pltpu.make_async_copy(
                sls[s].at[:, :, :, pl.ds(0, C)],
                out_hbm.at[pl.ds(0, FG), :, 0, :, pl.ds(0, C)],
                os_[s],
            ).wait()

        def chunk(ci, carry):
            b0 = b_w + ci * C
            pltpu.sync_copy(xT_hbm.at[:, pl.ds(b0, C)], xv)
            for j in range(R - 1):
                write_idx(j, j)
                issue_gather(j)

            def group(i, carry2):
                for j in range(R):
                    u = R * i + j
                    wait_gather(j)
                    unext = u + (R - 1)

                    @pl.when(unext < NU)
                    def _():
                        write_idx((j + R - 1) % R, unext)
                        issue_gather((j + R - 1) % R)

                    s = j % 2
                    if j < 2:
                        @pl.when(i > 0)
                        def _():
                            wait_out(s)
                    else:
                        wait_out(s)
                    transpose(j, s)
                    issue_out(s, FG * u, b0)
                return carry2

            lax.fori_loop(0, (NU - 1) // R, group, 0)
            wait_gather(0)
            wait_out(0)
            transpose(0, 0)
            issue_out(0, FG * (NU - 1), b0)
            wait_out(0)
            wait_out(1)
            return carry

        lax.fori_loop(0, NCH, chunk, 0)

    return k


_build_table = _build_table_kernel()
_gather = _gather_phase_kernel()


def kernel(inputs, feature_embedding, shared_embedding):
    xT = jnp.transpose(inputs)
    sh_pad = jnp.zeros((F * D_SH + L * 2,), jnp.float32)
    sh_pad = sh_pad.at[: F * D_SH].set(shared_embedding.reshape(F * D_SH))
    ft = jnp.transpose(feature_embedding)
    ft = jnp.pad(ft, ((0, D - D_FEAT), (0, VP - V)))
    feat5 = jnp.transpose(
        ft.reshape(4, 8, TB, 128), (0, 2, 1, 3)
    )
    ptab = _build_table(feat5, sh_pad)
    out5 = _gather(xT, ptab)
    outT = jnp.transpose(out5, (0, 1, 3, 2, 4)).reshape(F, D, B)
    return jnp.transpose(outT, (2, 0, 1))
